# baseline (speedup 1.0000x reference)
import functools

import jax
import jax.numpy as jnp
from jax import lax
from jax.experimental import pallas as pl
from jax.experimental.pallas import tpu as pltpu
from jax.experimental.pallas import tpu_sc as plsc

N = 10000
E = 320000
D = 128
DE = 16

NC = 2
NS = 16
NW = NC * NS
CHW = 80
CPW_FULL = 125
SPLIT = 63
NSLOT = 4
NSLOT_S = 2
NPAD = 10112
ROWS_PER_SUB = NPAD // NS
HALF_CPW = (38, 44, 43)
HALF_BLK = (1216, 1408, 1376)


@functools.cache
def _mesh():
    return plsc.VectorSubcoreMesh(
        core_axis_name="c", subcore_axis_name="s", num_cores=NC, num_subcores=NS
    )


def _gather_diff_body(cpw, src_hbm, dst_hbm, table_hbm, ntable_hbm, diff_hbm,
                      idx_s, idx_d, buf, sga, sgb, sout):
    cid = lax.axis_index("c")
    sid = lax.axis_index("s")
    wid = sid * NC + cid
    ebase = wid * (cpw * CHW)

    pltpu.sync_copy(src_hbm.at[wid], idx_s)
    pltpu.sync_copy(dst_hbm.at[wid], idx_d)

    def fire1(j, b):
        pltpu.async_copy(table_hbm.at[idx_s.at[j]], buf.at[b], sga.at[b])

    def fire2(j, b):
        pltpu.async_copy(ntable_hbm.at[idx_d.at[j]], buf.at[b], sgb.at[b],
                         add=True)

    fire1(0, 0)
    fire1(1, 1)
    pltpu.make_async_copy(table_hbm.at[idx_s.at[0]], buf.at[0], sga.at[0]).wait()
    fire2(0, 0)

    def visit(k, carry):
        b = lax.rem(k, NSLOT)
        b1 = lax.rem(k + 1, NSLOT)
        b2 = lax.rem(k + NSLOT // 2, NSLOT)

        @pl.when(k >= NSLOT // 2)
        def _drain_out():
            pltpu.make_async_copy(
                buf.at[b2], diff_hbm.at[pl.ds(0, CHW), :], sout.at[b2]).wait()

        @pl.when(k + NSLOT // 2 < cpw)
        def _fire_g1():
            fire1(k + NSLOT // 2, b2)

        @pl.when(k + 1 < cpw)
        def _fire_g2():
            pltpu.make_async_copy(
                table_hbm.at[idx_s.at[k]], buf.at[b1], sga.at[b1]).wait()
            fire2(k + 1, b1)

        pltpu.make_async_copy(
            ntable_hbm.at[idx_d.at[k]], buf.at[b], sgb.at[b]).wait()
        pltpu.async_copy(
            buf.at[b], diff_hbm.at[pl.ds(ebase + k * CHW, CHW), :], sout.at[b])
        return carry

    lax.fori_loop(0, cpw, visit, 0)
    for kk in range(cpw - NSLOT // 2, cpw):
        pltpu.make_async_copy(
            buf.at[kk % NSLOT], diff_hbm.at[pl.ds(0, CHW), :],
            sout.at[kk % NSLOT]).wait()


def _gather_diff(src3, dst3, node_feat, neg_node_feat, cpw):
    return pl.kernel(
        functools.partial(_gather_diff_body, cpw),
        out_type=jax.ShapeDtypeStruct((NW * cpw * CHW, D), jnp.float32),
        mesh=_mesh(),
        scratch_types=[
            pltpu.VMEM((cpw, CHW), jnp.int32),
            pltpu.VMEM((cpw, CHW), jnp.int32),
            pltpu.VMEM((NSLOT, CHW, D), jnp.float32),
            pltpu.SemaphoreType.DMA((NSLOT,)),
            pltpu.SemaphoreType.DMA((NSLOT,)),
            pltpu.SemaphoreType.DMA((NSLOT,)),
        ],
    )(src3, dst3, node_feat, neg_node_feat)


def _edge_mlp_body(x_ref, f_ref, w1m, b1m, w2, b2r, a2m, a2r, out_ref):
    x = x_ref[...]
    f = f_ref[...]
    dot = functools.partial(jnp.dot, preferred_element_type=jnp.float32)
    pre = dot(x, w1m[:D]) + dot(f, w1m[D:]) + b1m[...]
    h1 = jnp.maximum(pre[:, :D], 0.0)
    ah = jnp.maximum(pre[:, D:], 0.0)
    msg = dot(h1, w2[...]) + b2r[...]
    att = jax.nn.sigmoid(dot(ah, a2m[...]) + a2r[...])
    out_ref[...] = msg * att


def _edge_mlp(diff, edge_feat, w1m, b1m, w2, b2r, a2m, a2r, blk):
    ne = diff.shape[0]
    full = lambda shape: pl.BlockSpec(shape, lambda i: (0, 0))
    return pl.pallas_call(
        _edge_mlp_body,
        grid=(ne // blk,),
        in_specs=[
            pl.BlockSpec((blk, D), lambda i: (i, 0)),
            pl.BlockSpec((blk, DE), lambda i: (i, 0)),
            full((D + DE, 2 * D)), full((1, 2 * D)),
            full((D, D)), full((1, D)),
            full((D, D)), full((1, D)),
        ],
        out_specs=pl.BlockSpec((blk, D), lambda i: (i, 0)),
        out_shape=jax.ShapeDtypeStruct((ne, D), jnp.float32),
    )(diff, edge_feat, w1m, b1m, w2, b2r, a2m, a2r)


def _scatter_body(cpw, msg_hbm, dsti_hbm, zeros_hbm, out_hbm,
                  idx, mbuf, accum, srd, ssc):
    cid = lax.axis_index("c")
    sid = lax.axis_index("s")
    wid = sid * NC + cid
    ebase = wid * (cpw * CHW)
    rbase = sid * ROWS_PER_SUB
    rows = pl.ds(rbase, ROWS_PER_SUB)
    pltpu.sync_copy(zeros_hbm.at[rows, :], accum.at[rows, :])
    pltpu.sync_copy(dsti_hbm.at[wid], idx)
    plsc.subcore_barrier()

    def fire(j, b):
        pltpu.async_copy(
            msg_hbm.at[pl.ds(ebase + j * CHW, CHW), :], mbuf.at[b], srd.at[b])

    for j in range(NSLOT_S // 2):
        fire(j, j)

    def visit(k, carry):
        b = lax.rem(k, NSLOT_S)
        b2 = lax.rem(k + NSLOT_S // 2, NSLOT_S)

        @pl.when(k >= NSLOT_S // 2)
        def _drain_sc():
            pltpu.make_async_copy(
                mbuf.at[b2], accum.at[idx.at[k]], ssc.at[b2]).wait()

        @pl.when(k + NSLOT_S // 2 < cpw)
        def _refill():
            fire(k + NSLOT_S // 2, b2)

        pltpu.make_async_copy(
            msg_hbm.at[pl.ds(0, CHW), :], mbuf.at[b], srd.at[b]).wait()
        pltpu.async_copy(mbuf.at[b], accum.at[idx.at[k]], ssc.at[b], add=True)
        return carry

    lax.fori_loop(0, cpw, visit, 0)
    for kk in range(cpw - NSLOT_S // 2, cpw):
        pltpu.make_async_copy(
            mbuf.at[kk % NSLOT_S], accum.at[idx.at[kk]],
            ssc.at[kk % NSLOT_S]).wait()
    plsc.subcore_barrier()
    pltpu.sync_copy(accum.at[rows, :], out_hbm.at[cid, rows, :])


def _scatter_add(msg, dst3, zeros, cpw):
    return pl.kernel(
        functools.partial(_scatter_body, cpw),
        out_type=jax.ShapeDtypeStruct((NC, NPAD, D), jnp.float32),
        mesh=_mesh(),
        scratch_types=[
            pltpu.VMEM((cpw, CHW), jnp.int32),
            pltpu.VMEM((NSLOT_S, CHW, D), jnp.float32),
            pltpu.VMEM_SHARED((NPAD, D), jnp.float32),
            pltpu.SemaphoreType.DMA((NSLOT_S,)),
            pltpu.SemaphoreType.DMA((NSLOT_S,)),
        ],
    )(msg, dst3, zeros)


BLK_N = 2000


def _gru_body(p1_ref, p2_ref, p3_ref, h_ref, wih, bih, whh, bhh, out_ref):
    sm = (p1_ref[0] + p1_ref[1] + p2_ref[0] + p2_ref[1]
          + p3_ref[0] + p3_ref[1])
    h = h_ref[...]
    dot = functools.partial(jnp.dot, preferred_element_type=jnp.float32)
    gi = dot(sm, wih[...]) + bih[...]
    gh = dot(h, whh[...]) + bhh[...]
    r = jax.nn.sigmoid(gi[:, :D] + gh[:, :D])
    z = jax.nn.sigmoid(gi[:, D:2 * D] + gh[:, D:2 * D])
    n = jnp.tanh(gi[:, 2 * D:] + r * gh[:, 2 * D:])
    out_ref[...] = (1.0 - z) * n + z * h


def _gru(parts, h, wih, bih, whh, bhh):
    full = lambda shape: pl.BlockSpec(shape, lambda i: (0, 0))
    return pl.pallas_call(
        _gru_body,
        grid=(N // BLK_N,),
        in_specs=[
            pl.BlockSpec((NC, BLK_N, D), lambda i: (0, i, 0)),
            pl.BlockSpec((NC, BLK_N, D), lambda i: (0, i, 0)),
            pl.BlockSpec((NC, BLK_N, D), lambda i: (0, i, 0)),
            pl.BlockSpec((BLK_N, D), lambda i: (i, 0)),
            full((D, 3 * D)), full((1, 3 * D)),
            full((D, 3 * D)), full((1, 3 * D)),
        ],
        out_specs=pl.BlockSpec((BLK_N, D), lambda i: (i, 0)),
        out_shape=jax.ShapeDtypeStruct((N, D), jnp.float32),
    )(*parts, h, wih, bih, whh, bhh)


def kernel(node_feat, edge, edge_feat, W1, b1, W2, b2, A1, a1, A2, a2,
           W_ih, b_ih, W_hh, b_hh):
    nneg = jnp.negative(node_feat)
    zeros = jnp.zeros((NPAD, D), jnp.float32)
    w1m = jnp.concatenate(
        [jnp.concatenate([W1[:, :D].T, A1[:, :D].T], axis=1),
         jnp.concatenate([W1[:, D:].T, A1[:, D:].T], axis=1)], axis=0)
    b1m = jnp.concatenate([b1, a1])[None]
    parts = []
    e0 = 0
    for h in range(len(HALF_CPW)):
        cpw = HALF_CPW[h]
        ne = NW * cpw * CHW
        srch = lax.slice(edge, (e0, 0), (e0 + ne, 1)).reshape(NW, cpw, CHW)
        dsth = lax.slice(edge, (e0, 1), (e0 + ne, 2)).reshape(NW, cpw, CHW)
        diff = _gather_diff(srch, dsth, node_feat, nneg, cpw)
        msg = _edge_mlp(diff, edge_feat[e0:e0 + ne], w1m, b1m,
                        W2.T, b2[None], A2.T, a2[None], HALF_BLK[h])
        parts.append(_scatter_add(msg, dsth, zeros, cpw))
        e0 += ne
    return _gru(parts, node_feat,
                W_ih.T, b_ih[None], W_hh.T, b_hh[None])


# speedup vs baseline: 1.1188x; 1.1188x over previous
import functools

import jax
import jax.numpy as jnp
from jax import lax
from jax.experimental import pallas as pl
from jax.experimental.pallas import tpu as pltpu
from jax.experimental.pallas import tpu_sc as plsc

N = 10000
E = 320000
D = 128
DE = 16

NC = 2
NS = 16
NW = NC * NS
CHW = 80
CPW_FULL = 125
SPLIT = 63
NSLOT = 4
NSLOT_S = 2
NPAD = 10112
ROWS_PER_SUB = NPAD // NS
HALF_CPW = (63, 62)
HALF_BLK = (4032, 3968)


@functools.cache
def _mesh():
    return plsc.VectorSubcoreMesh(
        core_axis_name="c", subcore_axis_name="s", num_cores=NC, num_subcores=NS
    )


def _gather_diff_body(cpw, src_hbm, dst_hbm, table_hbm, ntable_hbm, diff_hbm,
                      idx_s, idx_d, buf, sga, sgb, sout):
    cid = lax.axis_index("c")
    sid = lax.axis_index("s")
    wid = sid * NC + cid
    ebase = wid * (cpw * CHW)

    pltpu.sync_copy(src_hbm.at[wid], idx_s)
    pltpu.sync_copy(dst_hbm.at[wid], idx_d)

    def fire1(j, b):
        pltpu.async_copy(table_hbm.at[idx_s.at[j]], buf.at[b], sga.at[b])

    def fire2(j, b):
        pltpu.async_copy(ntable_hbm.at[idx_d.at[j]], buf.at[b], sgb.at[b],
                         add=True)

    fire1(0, 0)
    fire1(1, 1)
    pltpu.make_async_copy(table_hbm.at[idx_s.at[0]], buf.at[0], sga.at[0]).wait()
    fire2(0, 0)

    def visit(k, carry):
        b = lax.rem(k, NSLOT)
        b1 = lax.rem(k + 1, NSLOT)
        b2 = lax.rem(k + NSLOT // 2, NSLOT)

        @pl.when(k >= NSLOT // 2)
        def _drain_out():
            pltpu.make_async_copy(
                buf.at[b2], diff_hbm.at[pl.ds(0, CHW), :], sout.at[b2]).wait()

        @pl.when(k + NSLOT // 2 < cpw)
        def _fire_g1():
            fire1(k + NSLOT // 2, b2)

        @pl.when(k + 1 < cpw)
        def _fire_g2():
            pltpu.make_async_copy(
                table_hbm.at[idx_s.at[k]], buf.at[b1], sga.at[b1]).wait()
            fire2(k + 1, b1)

        pltpu.make_async_copy(
            ntable_hbm.at[idx_d.at[k]], buf.at[b], sgb.at[b]).wait()
        pltpu.async_copy(
            buf.at[b], diff_hbm.at[pl.ds(ebase + k * CHW, CHW), :], sout.at[b])
        return carry

    lax.fori_loop(0, cpw, visit, 0)
    for kk in range(cpw - NSLOT // 2, cpw):
        pltpu.make_async_copy(
            buf.at[kk % NSLOT], diff_hbm.at[pl.ds(0, CHW), :],
            sout.at[kk % NSLOT]).wait()


def _gather_diff(src3, dst3, node_feat, neg_node_feat, cpw):
    return pl.kernel(
        functools.partial(_gather_diff_body, cpw),
        out_type=jax.ShapeDtypeStruct((NW * cpw * CHW, D), jnp.float32),
        mesh=_mesh(),
        scratch_types=[
            pltpu.VMEM((cpw, CHW), jnp.int32),
            pltpu.VMEM((cpw, CHW), jnp.int32),
            pltpu.VMEM((NSLOT, CHW, D), jnp.float32),
            pltpu.SemaphoreType.DMA((NSLOT,)),
            pltpu.SemaphoreType.DMA((NSLOT,)),
            pltpu.SemaphoreType.DMA((NSLOT,)),
        ],
    )(src3, dst3, node_feat, neg_node_feat)


def _edge_mlp_body(x_ref, f_ref, w1m, b1m, w2, b2r, a2m, a2r, out_ref):
    x = x_ref[...]
    f = f_ref[...]
    dot = functools.partial(jnp.dot, preferred_element_type=jnp.float32)
    pre = dot(x, w1m[:D]) + dot(f, w1m[D:]) + b1m[...]
    h1 = jnp.maximum(pre[:, :D], 0.0)
    ah = jnp.maximum(pre[:, D:], 0.0)
    msg = dot(h1, w2[...]) + b2r[...]
    att = jax.nn.sigmoid(dot(ah, a2m[...]) + a2r[...])
    out_ref[...] = msg * att


def _edge_mlp(diff, edge_feat, w1m, b1m, w2, b2r, a2m, a2r, blk):
    ne = diff.shape[0]
    full = lambda shape: pl.BlockSpec(shape, lambda i: (0, 0))
    return pl.pallas_call(
        _edge_mlp_body,
        grid=(ne // blk,),
        in_specs=[
            pl.BlockSpec((blk, D), lambda i: (i, 0)),
            pl.BlockSpec((blk, DE), lambda i: (i, 0)),
            full((D + DE, 2 * D)), full((1, 2 * D)),
            full((D, D)), full((1, D)),
            full((D, D)), full((1, D)),
        ],
        out_specs=pl.BlockSpec((blk, D), lambda i: (i, 0)),
        out_shape=jax.ShapeDtypeStruct((ne, D), jnp.float32),
    )(diff, edge_feat, w1m, b1m, w2, b2r, a2m, a2r)


def _scatter_body(cpw, msg_hbm, dsti_hbm, zeros_hbm, out_hbm,
                  idx, mbuf, accum, srd, ssc):
    cid = lax.axis_index("c")
    sid = lax.axis_index("s")
    wid = sid * NC + cid
    ebase = wid * (cpw * CHW)
    rbase = sid * ROWS_PER_SUB
    rows = pl.ds(rbase, ROWS_PER_SUB)
    pltpu.sync_copy(zeros_hbm.at[rows, :], accum.at[rows, :])
    pltpu.sync_copy(dsti_hbm.at[wid], idx)
    plsc.subcore_barrier()

    def fire(j, b):
        pltpu.async_copy(
            msg_hbm.at[pl.ds(ebase + j * CHW, CHW), :], mbuf.at[b], srd.at[b])

    for j in range(NSLOT_S // 2):
        fire(j, j)

    def visit(k, carry):
        b = lax.rem(k, NSLOT_S)
        b2 = lax.rem(k + NSLOT_S // 2, NSLOT_S)

        @pl.when(k >= NSLOT_S // 2)
        def _drain_sc():
            pltpu.make_async_copy(
                mbuf.at[b2], accum.at[idx.at[k]], ssc.at[b2]).wait()

        @pl.when(k + NSLOT_S // 2 < cpw)
        def _refill():
            fire(k + NSLOT_S // 2, b2)

        pltpu.make_async_copy(
            msg_hbm.at[pl.ds(0, CHW), :], mbuf.at[b], srd.at[b]).wait()
        pltpu.async_copy(mbuf.at[b], accum.at[idx.at[k]], ssc.at[b], add=True)
        return carry

    lax.fori_loop(0, cpw, visit, 0)
    for kk in range(cpw - NSLOT_S // 2, cpw):
        pltpu.make_async_copy(
            mbuf.at[kk % NSLOT_S], accum.at[idx.at[kk]],
            ssc.at[kk % NSLOT_S]).wait()
    plsc.subcore_barrier()
    pltpu.sync_copy(accum.at[rows, :], out_hbm.at[cid, rows, :])


def _scatter_add(msg, dst3, zeros, cpw):
    return pl.kernel(
        functools.partial(_scatter_body, cpw),
        out_type=jax.ShapeDtypeStruct((NC, NPAD, D), jnp.float32),
        mesh=_mesh(),
        scratch_types=[
            pltpu.VMEM((cpw, CHW), jnp.int32),
            pltpu.VMEM((NSLOT_S, CHW, D), jnp.float32),
            pltpu.VMEM_SHARED((NPAD, D), jnp.float32),
            pltpu.SemaphoreType.DMA((NSLOT_S,)),
            pltpu.SemaphoreType.DMA((NSLOT_S,)),
        ],
    )(msg, dst3, zeros)


BLK_N = 2000


def _gru_body(p1_ref, p2_ref, h_ref, wih, bih, whh, bhh, out_ref):
    sm = p1_ref[0] + p1_ref[1] + p2_ref[0] + p2_ref[1]
    h = h_ref[...]
    dot = functools.partial(jnp.dot, preferred_element_type=jnp.float32)
    gi = dot(sm, wih[...]) + bih[...]
    gh = dot(h, whh[...]) + bhh[...]
    r = jax.nn.sigmoid(gi[:, :D] + gh[:, :D])
    z = jax.nn.sigmoid(gi[:, D:2 * D] + gh[:, D:2 * D])
    n = jnp.tanh(gi[:, 2 * D:] + r * gh[:, 2 * D:])
    out_ref[...] = (1.0 - z) * n + z * h


def _gru(parts, h, wih, bih, whh, bhh):
    full = lambda shape: pl.BlockSpec(shape, lambda i: (0, 0))
    return pl.pallas_call(
        _gru_body,
        grid=(N // BLK_N,),
        in_specs=[
            pl.BlockSpec((NC, BLK_N, D), lambda i: (0, i, 0)),
            pl.BlockSpec((NC, BLK_N, D), lambda i: (0, i, 0)),
            pl.BlockSpec((BLK_N, D), lambda i: (i, 0)),
            full((D, 3 * D)), full((1, 3 * D)),
            full((D, 3 * D)), full((1, 3 * D)),
        ],
        out_specs=pl.BlockSpec((BLK_N, D), lambda i: (i, 0)),
        out_shape=jax.ShapeDtypeStruct((N, D), jnp.float32),
    )(*parts, h, wih, bih, whh, bhh)


def kernel(node_feat, edge, edge_feat, W1, b1, W2, b2, A1, a1, A2, a2,
           W_ih, b_ih, W_hh, b_hh):
    nneg = jnp.negative(node_feat)
    zeros = jnp.zeros((NPAD, D), jnp.float32)
    w1m = jnp.concatenate(
        [jnp.concatenate([W1[:, :D].T, A1[:, :D].T], axis=1),
         jnp.concatenate([W1[:, D:].T, A1[:, D:].T], axis=1)], axis=0)
    b1m = jnp.concatenate([b1, a1])[None]
    parts = []
    e0 = 0
    for h in range(len(HALF_CPW)):
        cpw = HALF_CPW[h]
        ne = NW * cpw * CHW
        srch = lax.slice(edge, (e0, 0), (e0 + ne, 1)).reshape(NW, cpw, CHW)
        dsth = lax.slice(edge, (e0, 1), (e0 + ne, 2)).reshape(NW, cpw, CHW)
        diff = _gather_diff(srch, dsth, node_feat, nneg, cpw)
        msg = _edge_mlp(diff, edge_feat[e0:e0 + ne], w1m, b1m,
                        W2.T, b2[None], A2.T, a2[None], HALF_BLK[h])
        parts.append(_scatter_add(msg, dsth, zeros, cpw))
        e0 += ne
    return _gru(parts, node_feat,
                W_ih.T, b_ih[None], W_hh.T, b_hh[None])


# speedup vs baseline: 1.1541x; 1.0315x over previous
import functools

import jax
import jax.numpy as jnp
from jax import lax
from jax.experimental import pallas as pl
from jax.experimental.pallas import tpu as pltpu
from jax.experimental.pallas import tpu_sc as plsc

N = 10000
E = 320000
D = 128
DE = 16

NC = 2
NS = 16
NW = NC * NS
CHW = 80
CPW_FULL = 125
SPLIT = 63
NSLOT = 4
NSLOT_S = 2
NPAD = 10112
ROWS_PER_SUB = NPAD // NS
HALF_CPW = (63, 62)
HALF_BLK = (8064, 7936)


@functools.cache
def _mesh():
    return plsc.VectorSubcoreMesh(
        core_axis_name="c", subcore_axis_name="s", num_cores=NC, num_subcores=NS
    )


def _gather_diff_body(cpw, src_hbm, dst_hbm, table_hbm, ntable_hbm, diff_hbm,
                      idx_s, idx_d, buf, sga, sgb, sout):
    cid = lax.axis_index("c")
    sid = lax.axis_index("s")
    wid = sid * NC + cid
    ebase = wid * (cpw * CHW)

    pltpu.sync_copy(src_hbm.at[wid], idx_s)
    pltpu.sync_copy(dst_hbm.at[wid], idx_d)

    def fire1(j, b):
        pltpu.async_copy(table_hbm.at[idx_s.at[j]], buf.at[b], sga.at[b])

    def fire2(j, b):
        pltpu.async_copy(ntable_hbm.at[idx_d.at[j]], buf.at[b], sgb.at[b],
                         add=True)

    fire1(0, 0)
    fire1(1, 1)
    pltpu.make_async_copy(table_hbm.at[idx_s.at[0]], buf.at[0], sga.at[0]).wait()
    fire2(0, 0)

    def visit(k, carry):
        b = lax.rem(k, NSLOT)
        b1 = lax.rem(k + 1, NSLOT)
        b2 = lax.rem(k + NSLOT // 2, NSLOT)

        @pl.when(k >= NSLOT // 2)
        def _drain_out():
            pltpu.make_async_copy(
                buf.at[b2], diff_hbm.at[pl.ds(0, CHW), :], sout.at[b2]).wait()

        @pl.when(k + NSLOT // 2 < cpw)
        def _fire_g1():
            fire1(k + NSLOT // 2, b2)

        @pl.when(k + 1 < cpw)
        def _fire_g2():
            pltpu.make_async_copy(
                table_hbm.at[idx_s.at[k]], buf.at[b1], sga.at[b1]).wait()
            fire2(k + 1, b1)

        pltpu.make_async_copy(
            ntable_hbm.at[idx_d.at[k]], buf.at[b], sgb.at[b]).wait()
        pltpu.async_copy(
            buf.at[b], diff_hbm.at[pl.ds(ebase + k * CHW, CHW), :], sout.at[b])
        return carry

    lax.fori_loop(0, cpw, visit, 0)
    for kk in range(cpw - NSLOT // 2, cpw):
        pltpu.make_async_copy(
            buf.at[kk % NSLOT], diff_hbm.at[pl.ds(0, CHW), :],
            sout.at[kk % NSLOT]).wait()


def _gather_diff(src3, dst3, node_feat, neg_node_feat, cpw):
    return pl.kernel(
        functools.partial(_gather_diff_body, cpw),
        out_type=jax.ShapeDtypeStruct((NW * cpw * CHW, D), jnp.float32),
        mesh=_mesh(),
        scratch_types=[
            pltpu.VMEM((cpw, CHW), jnp.int32),
            pltpu.VMEM((cpw, CHW), jnp.int32),
            pltpu.VMEM((NSLOT, CHW, D), jnp.float32),
            pltpu.SemaphoreType.DMA((NSLOT,)),
            pltpu.SemaphoreType.DMA((NSLOT,)),
            pltpu.SemaphoreType.DMA((NSLOT,)),
        ],
    )(src3, dst3, node_feat, neg_node_feat)


def _edge_mlp_body(x_ref, f_ref, w1m, b1m, w2, b2r, a2m, a2r, out_ref):
    x = x_ref[...]
    f = f_ref[...]
    dot = functools.partial(jnp.dot, preferred_element_type=jnp.float32)
    pre = dot(x, w1m[:D]) + dot(f, w1m[D:]) + b1m[...]
    h1 = jnp.maximum(pre[:, :D], 0.0)
    ah = jnp.maximum(pre[:, D:], 0.0)
    msg = dot(h1, w2[...]) + b2r[...]
    att = jax.nn.sigmoid(dot(ah, a2m[...]) + a2r[...])
    out_ref[...] = msg * att


def _edge_mlp(diff, edge_feat, w1m, b1m, w2, b2r, a2m, a2r, blk):
    ne = diff.shape[0]
    full = lambda shape: pl.BlockSpec(shape, lambda i: (0, 0))
    return pl.pallas_call(
        _edge_mlp_body,
        grid=(ne // blk,),
        in_specs=[
            pl.BlockSpec((blk, D), lambda i: (i, 0)),
            pl.BlockSpec((blk, DE), lambda i: (i, 0)),
            full((D + DE, 2 * D)), full((1, 2 * D)),
            full((D, D)), full((1, D)),
            full((D, D)), full((1, D)),
        ],
        out_specs=pl.BlockSpec((blk, D), lambda i: (i, 0)),
        out_shape=jax.ShapeDtypeStruct((ne, D), jnp.float32),
    )(diff, edge_feat, w1m, b1m, w2, b2r, a2m, a2r)


def _scatter_body(cpw, msg_hbm, dsti_hbm, zeros_hbm, out_hbm,
                  idx, mbuf, accum, srd, ssc):
    cid = lax.axis_index("c")
    sid = lax.axis_index("s")
    wid = sid * NC + cid
    ebase = wid * (cpw * CHW)
    rbase = sid * ROWS_PER_SUB
    rows = pl.ds(rbase, ROWS_PER_SUB)
    pltpu.sync_copy(zeros_hbm.at[rows, :], accum.at[rows, :])
    pltpu.sync_copy(dsti_hbm.at[wid], idx)
    plsc.subcore_barrier()

    def fire(j, b):
        pltpu.async_copy(
            msg_hbm.at[pl.ds(ebase + j * CHW, CHW), :], mbuf.at[b], srd.at[b])

    for j in range(NSLOT_S // 2):
        fire(j, j)

    def visit(k, carry):
        b = lax.rem(k, NSLOT_S)
        b2 = lax.rem(k + NSLOT_S // 2, NSLOT_S)

        @pl.when(k >= NSLOT_S // 2)
        def _drain_sc():
            pltpu.make_async_copy(
                mbuf.at[b2], accum.at[idx.at[k]], ssc.at[b2]).wait()

        @pl.when(k + NSLOT_S // 2 < cpw)
        def _refill():
            fire(k + NSLOT_S // 2, b2)

        pltpu.make_async_copy(
            msg_hbm.at[pl.ds(0, CHW), :], mbuf.at[b], srd.at[b]).wait()
        pltpu.async_copy(mbuf.at[b], accum.at[idx.at[k]], ssc.at[b], add=True)
        return carry

    lax.fori_loop(0, cpw, visit, 0)
    for kk in range(cpw - NSLOT_S // 2, cpw):
        pltpu.make_async_copy(
            mbuf.at[kk % NSLOT_S], accum.at[idx.at[kk]],
            ssc.at[kk % NSLOT_S]).wait()
    plsc.subcore_barrier()
    pltpu.sync_copy(accum.at[rows, :], out_hbm.at[cid, rows, :])


def _scatter_add(msg, dst3, zeros, cpw):
    return pl.kernel(
        functools.partial(_scatter_body, cpw),
        out_type=jax.ShapeDtypeStruct((NC, NPAD, D), jnp.float32),
        mesh=_mesh(),
        scratch_types=[
            pltpu.VMEM((cpw, CHW), jnp.int32),
            pltpu.VMEM((NSLOT_S, CHW, D), jnp.float32),
            pltpu.VMEM_SHARED((NPAD, D), jnp.float32),
            pltpu.SemaphoreType.DMA((NSLOT_S,)),
            pltpu.SemaphoreType.DMA((NSLOT_S,)),
        ],
    )(msg, dst3, zeros)


BLK_N = 2000


def _gru_body(p1_ref, p2_ref, h_ref, wih, bih, whh, bhh, out_ref):
    sm = p1_ref[0] + p1_ref[1] + p2_ref[0] + p2_ref[1]
    h = h_ref[...]
    dot = functools.partial(jnp.dot, preferred_element_type=jnp.float32)
    gi = dot(sm, wih[...]) + bih[...]
    gh = dot(h, whh[...]) + bhh[...]
    r = jax.nn.sigmoid(gi[:, :D] + gh[:, :D])
    z = jax.nn.sigmoid(gi[:, D:2 * D] + gh[:, D:2 * D])
    n = jnp.tanh(gi[:, 2 * D:] + r * gh[:, 2 * D:])
    out_ref[...] = (1.0 - z) * n + z * h


def _gru(parts, h, wih, bih, whh, bhh):
    full = lambda shape: pl.BlockSpec(shape, lambda i: (0, 0))
    return pl.pallas_call(
        _gru_body,
        grid=(N // BLK_N,),
        in_specs=[
            pl.BlockSpec((NC, BLK_N, D), lambda i: (0, i, 0)),
            pl.BlockSpec((NC, BLK_N, D), lambda i: (0, i, 0)),
            pl.BlockSpec((BLK_N, D), lambda i: (i, 0)),
            full((D, 3 * D)), full((1, 3 * D)),
            full((D, 3 * D)), full((1, 3 * D)),
        ],
        out_specs=pl.BlockSpec((BLK_N, D), lambda i: (i, 0)),
        out_shape=jax.ShapeDtypeStruct((N, D), jnp.float32),
    )(*parts, h, wih, bih, whh, bhh)


def kernel(node_feat, edge, edge_feat, W1, b1, W2, b2, A1, a1, A2, a2,
           W_ih, b_ih, W_hh, b_hh):
    nneg = jnp.negative(node_feat)
    zeros = jnp.zeros((NPAD, D), jnp.float32)
    w1m = jnp.concatenate(
        [jnp.concatenate([W1[:, :D].T, A1[:, :D].T], axis=1),
         jnp.concatenate([W1[:, D:].T, A1[:, D:].T], axis=1)], axis=0)
    b1m = jnp.concatenate([b1, a1])[None]
    parts = []
    e0 = 0
    for h in range(len(HALF_CPW)):
        cpw = HALF_CPW[h]
        ne = NW * cpw * CHW
        srch = lax.slice(edge, (e0, 0), (e0 + ne, 1)).reshape(NW, cpw, CHW)
        dsth = lax.slice(edge, (e0, 1), (e0 + ne, 2)).reshape(NW, cpw, CHW)
        diff = _gather_diff(srch, dsth, node_feat, nneg, cpw)
        msg = _edge_mlp(diff, edge_feat[e0:e0 + ne], w1m, b1m,
                        W2.T, b2[None], A2.T, a2[None], HALF_BLK[h])
        parts.append(_scatter_add(msg, dsth, zeros, cpw))
        e0 += ne
    return _gru(parts, node_feat,
                W_ih.T, b_ih[None], W_hh.T, b_hh[None])
